# baseline (speedup 1.0000x reference)
import jax
import jax.numpy as jnp
from jax.experimental import pallas as pl
from jax.experimental.pallas import tpu as pltpu

_ROWS = 16384
_FEAT = 768


def _concat_dma_kernel(v_ref, t_ref, o_ref, sem_v, sem_t):
    cv = pltpu.make_async_copy(v_ref, o_ref.at[:, 0:_FEAT], sem_v)
    ct = pltpu.make_async_copy(t_ref, o_ref.at[:, _FEAT : 2 * _FEAT], sem_t)
    cv.start()
    ct.start()
    cv.wait()
    ct.wait()


def kernel(vision_input, text_input):
    out_shape = jax.ShapeDtypeStruct((_ROWS, 2 * _FEAT), vision_input.dtype)
    return pl.pallas_call(
        _concat_dma_kernel,
        out_shape=out_shape,
        in_specs=[
            pl.BlockSpec(memory_space=pltpu.MemorySpace.HBM),
            pl.BlockSpec(memory_space=pltpu.MemorySpace.HBM),
        ],
        out_specs=pl.BlockSpec(memory_space=pltpu.MemorySpace.HBM),
        scratch_shapes=[pltpu.SemaphoreType.DMA, pltpu.SemaphoreType.DMA],
    )(vision_input, text_input)


# speedup vs baseline: 48.5364x; 48.5364x over previous
import jax
import jax.numpy as jnp
from jax.experimental import pallas as pl
from jax.experimental.pallas import tpu as pltpu

_ROWS = 16384
_FEAT = 768
_BM = 1024


def _concat_kernel(v_ref, t_ref, o_ref):
    o_ref[:, 0:_FEAT] = v_ref[...]
    o_ref[:, _FEAT : 2 * _FEAT] = t_ref[...]


def kernel(vision_input, text_input):
    out_shape = jax.ShapeDtypeStruct((_ROWS, 2 * _FEAT), vision_input.dtype)
    return pl.pallas_call(
        _concat_kernel,
        grid=(_ROWS // _BM,),
        in_specs=[
            pl.BlockSpec((_BM, _FEAT), lambda i: (i, 0)),
            pl.BlockSpec((_BM, _FEAT), lambda i: (i, 0)),
        ],
        out_specs=pl.BlockSpec((_BM, 2 * _FEAT), lambda i: (i, 0)),
        out_shape=out_shape,
    )(vision_input, text_input)


# speedup vs baseline: 49.3146x; 1.0160x over previous
import jax
import jax.numpy as jnp
from jax.experimental import pallas as pl
from jax.experimental.pallas import tpu as pltpu

_ROWS = 16384
_FEAT = 768
_BM = 2048


def _concat_kernel(v_ref, t_ref, o_ref):
    o_ref[:, 0:_FEAT] = v_ref[...]
    o_ref[:, _FEAT : 2 * _FEAT] = t_ref[...]


def kernel(vision_input, text_input):
    out_shape = jax.ShapeDtypeStruct((_ROWS, 2 * _FEAT), vision_input.dtype)
    return pl.pallas_call(
        _concat_kernel,
        grid=(_ROWS // _BM,),
        in_specs=[
            pl.BlockSpec((_BM, _FEAT), lambda i: (i, 0)),
            pl.BlockSpec((_BM, _FEAT), lambda i: (i, 0)),
        ],
        out_specs=pl.BlockSpec((_BM, 2 * _FEAT), lambda i: (i, 0)),
        out_shape=out_shape,
    )(vision_input, text_input)
